# baseline (speedup 1.0000x reference)
import functools
import math

import jax
import jax.numpy as jnp
from jax import lax
from jax.experimental import pallas as pl
from jax.experimental.pallas import tpu as pltpu
from jax.experimental.pallas import tpu_sc as plsc

N = 10000
E = 160000
F = 256
H = 2
FH = F // H

NUM_TILES = 16
EPT = E // NUM_TILES
CHUNK = 32
NFULL = EPT // CHUNK
TAIL_BASE = EPT - CHUNK
NCHUNKS = NFULL + 1
NGROUP = CHUNK // 16
ROWS_PER_TILE = 624
ROWS_REM = N - NUM_TILES * ROWS_PER_TILE
NSUB = FH // 16


_BN = 1000


def _qkv_body(x_ref, wq_ref, wk_ref, wv_ref, q_ref, k_ref, v_ref):
    xb = x_ref[...]
    dn = (((1,), (1,)), ((), ()))
    q_ref[0] = lax.dot_general(xb, wq_ref[0], dn, preferred_element_type=jnp.float32)
    k_ref[0] = lax.dot_general(xb, wk_ref[0], dn, preferred_element_type=jnp.float32)
    v_ref[0] = lax.dot_general(xb, wv_ref[0], dn, preferred_element_type=jnp.float32)


def _qkv_project(x, Wq, Wk, Wv):
    grid = (H, N // _BN)
    out = jax.ShapeDtypeStruct((H, N, FH), jnp.float32)
    return pl.pallas_call(
        _qkv_body,
        grid=grid,
        in_specs=[
            pl.BlockSpec((_BN, FH), lambda h, i: (i, h)),
            pl.BlockSpec((1, FH, FH), lambda h, i: (h, 0, 0)),
            pl.BlockSpec((1, FH, FH), lambda h, i: (h, 0, 0)),
            pl.BlockSpec((1, FH, FH), lambda h, i: (h, 0, 0)),
        ],
        out_specs=[
            pl.BlockSpec((1, _BN, FH), lambda h, i: (h, i, 0)),
            pl.BlockSpec((1, _BN, FH), lambda h, i: (h, i, 0)),
            pl.BlockSpec((1, _BN, FH), lambda h, i: (h, i, 0)),
        ],
        out_shape=[out, out, out],
    )(x, Wq, Wk, Wv)


_GATHER_DN = lax.GatherDimensionNumbers(
    offset_dims=(), collapsed_slice_dims=(0,), start_index_map=(0,))


def _lane_shuffle(v, perm):
    return lax.gather(v, perm.reshape(16, 1), _GATHER_DN, (1,),
                      mode=lax.GatherScatterMode.PROMISE_IN_BOUNDS)


def _lane_total(v):
    iota = lax.iota(jnp.int32, 16)
    for sh in (8, 4, 2, 1):
        v = v + _lane_shuffle(v, iota ^ sh)
    return v


def _edge_body(q_hbm, k_hbm, v_hbm, w_hbm, phi_hbm, ii_hbm, ij_hbm, out_hbm,
               ii8, ij8, phib, qb3, kb3, vb3, wb3, cb3,
               ysh, sem_i, gsem, ssem):
    c = lax.axis_index("c")
    s = lax.axis_index("s")
    t0 = s * EPT

    zb = qb3.at[0]

    def _zrow(r, _):
        for t in range(NSUB):
            zb[r, pl.ds(t * 16, 16)] = jnp.zeros((16,), jnp.float32)
        return 0

    lax.fori_loop(0, CHUNK, _zrow, 0)
    r0 = s * ROWS_PER_TILE
    nz = ROWS_PER_TILE // CHUNK
    zcps = []
    for z in range(nz):
        zcps.append(pltpu.async_copy(zb, ysh.at[pl.ds(r0 + z * CHUNK, CHUNK)], sem_i))
    rem = ROWS_PER_TILE - nz * CHUNK
    if rem:
        zcps.append(pltpu.async_copy(zb.at[pl.ds(0, rem)],
                                     ysh.at[pl.ds(r0 + nz * CHUNK, rem)], sem_i))

    @pl.when(s == 0)
    def _zero_tail():
        pltpu.async_copy(zb.at[pl.ds(0, ROWS_REM)],
                         ysh.at[pl.ds(NUM_TILES * ROWS_PER_TILE, ROWS_REM)],
                         sem_i).wait()

    for cp in zcps:
        cp.wait()
    plsc.subcore_barrier()

    def chunk_base(nn):
        return t0 + jnp.where(nn >= NFULL, TAIL_BASE, nn * CHUNK)

    def issue_idx(b, r):
        pltpu.async_copy(ii_hbm.at[pl.ds(b, CHUNK)], ii8.at[r], sem_i)
        pltpu.async_copy(ij_hbm.at[pl.ds(b, CHUNK)], ij8.at[r], sem_i)
        pltpu.async_copy(phi_hbm.at[pl.ds(b, CHUNK)], phib.at[r], sem_i)

    def wait_idx():
        pltpu.make_async_copy(ii_hbm.at[pl.ds(0, CHUNK)], ii8.at[0], sem_i).wait()
        pltpu.make_async_copy(ij_hbm.at[pl.ds(0, CHUNK)], ij8.at[0], sem_i).wait()
        pltpu.make_async_copy(phi_hbm.at[pl.ds(0, CHUNK)], phib.at[0], sem_i).wait()

    def issue_gathers(b, u, r):
        for hh in range(2):
            hs = pl.ds(hh * 16, 16)
            pltpu.async_copy(q_hbm.at[c].at[ii8.at[r, hs]],
                             qb3.at[u].at[hs], gsem.at[u])
            pltpu.async_copy(k_hbm.at[c].at[ij8.at[r, hs]],
                             kb3.at[u].at[hs], gsem.at[u])
            pltpu.async_copy(v_hbm.at[c].at[ij8.at[r, hs]],
                             vb3.at[u].at[hs], gsem.at[u])
        pltpu.async_copy(w_hbm.at[pl.ds(b, CHUNK), pl.ds(c * FH, FH)], wb3.at[u],
                         gsem.at[u])

    def wait_gathers(u):
        for hh in range(2):
            hs = pl.ds(hh * 16, 16)
            pltpu.make_async_copy(q_hbm.at[c].at[ii8.at[0, hs]],
                                  qb3.at[u].at[hs], gsem.at[u]).wait()
            pltpu.make_async_copy(k_hbm.at[c].at[ij8.at[0, hs]],
                                  kb3.at[u].at[hs], gsem.at[u]).wait()
            pltpu.make_async_copy(v_hbm.at[c].at[ij8.at[0, hs]],
                                  vb3.at[u].at[hs], gsem.at[u]).wait()
        pltpu.make_async_copy(w_hbm.at[pl.ds(0, CHUNK), pl.ds(0, FH)], wb3.at[u],
                              gsem.at[u]).wait()

    def compute(u, r):
        @plsc.parallel_loop(0, NGROUP, 1)
        def _group(g):
            pv = phib[r, pl.ds(g * 16, 16)]
            for j in range(16):
                e = g * 16 + j
                acc = jnp.zeros((16,), jnp.float32)
                for t in range(NSUB):
                    sl = pl.ds(t * 16, 16)
                    acc = acc + qb3[u, e, sl] * wb3[u, e, sl] * kb3[u, e, sl]
                alpha = _lane_total(acc) * pv[j]
                for t in range(NSUB):
                    sl = pl.ds(t * 16, 16)
                    cb3[u, e, sl] = alpha * vb3[u, e, sl]

    def issue_scatter(u, r):
        for hh in range(2):
            hs = pl.ds(hh * 16, 16)
            pltpu.async_copy(cb3.at[u].at[hs], ysh.at[ii8.at[r, hs]],
                             ssem.at[u], add=True)

    def wait_scatter(u):
        for hh in range(2):
            hs = pl.ds(hh * 16, 16)
            pltpu.make_async_copy(cb3.at[u].at[hs], ysh.at[ii8.at[0, hs]],
                                  ssem.at[u]).wait()

    pltpu.sync_copy(ii_hbm.at[pl.ds(t0, CHUNK)], ii8.at[0])
    pltpu.sync_copy(ij_hbm.at[pl.ds(t0, CHUNK)], ij8.at[0])
    pltpu.sync_copy(phi_hbm.at[pl.ds(t0, CHUNK)], phib.at[0])
    issue_gathers(t0, 0, 0)
    issue_idx(t0 + CHUNK, 1)

    def _step(n, _):
        u = lax.rem(n, 2)
        r = lax.rem(n, 8)
        wait_idx()
        issue_idx(chunk_base(n + 2), lax.rem(n + 2, 8))
        issue_gathers(chunk_base(n + 1), lax.rem(n + 1, 2), lax.rem(n + 1, 8))
        wait_gathers(u)

        @pl.when(n >= 2)
        def _ws():
            wait_scatter(u)

        compute(u, r)

        @pl.when(n == NFULL)
        def _mask_tail():
            def _zdup(rr, _):
                for t in range(NSUB):
                    cb3[u, rr, pl.ds(t * 16, 16)] = jnp.zeros((16,), jnp.float32)
                return 0

            lax.fori_loop(0, CHUNK - 16, _zdup, 0)

        issue_scatter(u, r)
        return 0

    lax.fori_loop(0, NCHUNKS, _step, 0)

    wait_idx()
    wait_gathers(1)
    wait_scatter(1)
    wait_scatter(0)
    plsc.subcore_barrier()

    done = 0
    while done < ROWS_PER_TILE:
        sz = min(128, ROWS_PER_TILE - done)
        pltpu.sync_copy(ysh.at[pl.ds(r0 + done, sz)],
                        out_hbm.at[pl.ds(r0 + done, sz), pl.ds(c * FH, FH)])
        done += sz

    @pl.when(s == 0)
    def _write_tail():
        tb = NUM_TILES * ROWS_PER_TILE
        pltpu.sync_copy(ysh.at[pl.ds(tb, ROWS_REM)],
                        out_hbm.at[pl.ds(tb, ROWS_REM), pl.ds(c * FH, FH)])


def _edge_kernel(q_all, k_all, v_all, w_ij, phi_s, ii, ij):
    mesh = plsc.VectorSubcoreMesh(core_axis_name="c", subcore_axis_name="s")
    run = functools.partial(
        pl.kernel,
        out_type=jax.ShapeDtypeStruct((N, F), jnp.float32),
        mesh=mesh,
        scratch_types=[
            pltpu.VMEM((8, CHUNK), jnp.int32),
            pltpu.VMEM((8, CHUNK), jnp.int32),
            pltpu.VMEM((8, CHUNK), jnp.float32),
            pltpu.VMEM((2, CHUNK, FH), jnp.float32),
            pltpu.VMEM((2, CHUNK, FH), jnp.float32),
            pltpu.VMEM((2, CHUNK, FH), jnp.float32),
            pltpu.VMEM((2, CHUNK, FH), jnp.float32),
            pltpu.VMEM((2, CHUNK, FH), jnp.float32),
            pltpu.VMEM_SHARED((N, FH), jnp.float32),
            pltpu.SemaphoreType.DMA,
            pltpu.SemaphoreType.DMA((2,)),
            pltpu.SemaphoreType.DMA((2,)),
        ],
    )(_edge_body)
    return run(q_all, k_all, v_all, w_ij, phi_s, ii, ij)


def kernel(x, w_ij, phi_r_cut, idx_i, idx_j, Wq, Wk, Wv):
    q_all, k_all, v_all = _qkv_project(x, Wq, Wk, Wv)
    phi_s = phi_r_cut[:, 0] * jnp.float32(1.0 / math.sqrt(FH))
    ii = idx_i.astype(jnp.int32)
    ij = idx_j.astype(jnp.int32)
    return _edge_kernel(q_all, k_all, v_all, w_ij, phi_s, ii, ij)


# speedup vs baseline: 1.0057x; 1.0057x over previous
import functools
import math

import jax
import jax.numpy as jnp
from jax import lax
from jax.experimental import pallas as pl
from jax.experimental.pallas import tpu as pltpu
from jax.experimental.pallas import tpu_sc as plsc

N = 10000
E = 160000
F = 256
H = 2
FH = F // H

NUM_TILES = 16
EPT = E // NUM_TILES
CHUNK = 32
NFULL = EPT // CHUNK
TAIL_BASE = EPT - CHUNK
NCHUNKS = NFULL + 1
NGROUP = CHUNK // 16
ROWS_PER_TILE = 624
ROWS_REM = N - NUM_TILES * ROWS_PER_TILE
NSUB = FH // 16


_BN = 1000


def _qkv_body(x_ref, wq_ref, wk_ref, wv_ref, q_ref, k_ref, v_ref):
    xb = x_ref[...]
    dn = (((1,), (1,)), ((), ()))
    q_ref[0] = lax.dot_general(xb, wq_ref[0], dn, preferred_element_type=jnp.float32)
    k_ref[0] = lax.dot_general(xb, wk_ref[0], dn, preferred_element_type=jnp.float32)
    v_ref[0] = lax.dot_general(xb, wv_ref[0], dn, preferred_element_type=jnp.float32)


def _qkv_project(x, Wq, Wk, Wv):
    grid = (H, N // _BN)
    out = jax.ShapeDtypeStruct((H, N, FH), jnp.float32)
    return pl.pallas_call(
        _qkv_body,
        grid=grid,
        in_specs=[
            pl.BlockSpec((_BN, FH), lambda h, i: (i, h)),
            pl.BlockSpec((1, FH, FH), lambda h, i: (h, 0, 0)),
            pl.BlockSpec((1, FH, FH), lambda h, i: (h, 0, 0)),
            pl.BlockSpec((1, FH, FH), lambda h, i: (h, 0, 0)),
        ],
        out_specs=[
            pl.BlockSpec((1, _BN, FH), lambda h, i: (h, i, 0)),
            pl.BlockSpec((1, _BN, FH), lambda h, i: (h, i, 0)),
            pl.BlockSpec((1, _BN, FH), lambda h, i: (h, i, 0)),
        ],
        out_shape=[out, out, out],
    )(x, Wq, Wk, Wv)


_GATHER_DN = lax.GatherDimensionNumbers(
    offset_dims=(), collapsed_slice_dims=(0,), start_index_map=(0,))


def _lane_shuffle(v, perm):
    return lax.gather(v, perm.reshape(16, 1), _GATHER_DN, (1,),
                      mode=lax.GatherScatterMode.PROMISE_IN_BOUNDS)


def _lane_total(v):
    iota = lax.iota(jnp.int32, 16)
    for sh in (8, 4, 2, 1):
        v = v + _lane_shuffle(v, iota ^ sh)
    return v


def _edge_body(q_hbm, k_hbm, v_hbm, w_hbm, phi_hbm, ii_hbm, ij_hbm, out_hbm,
               ii8, ij8, phib, qb3, kb3, vb3, wb3, cb3,
               ysh, sem_i, gsem, ssem):
    c = lax.axis_index("c")
    s = lax.axis_index("s")
    t0 = s * EPT

    zb = qb3.at[0]

    def _zrow(r, _):
        for t in range(NSUB):
            zb[r, pl.ds(t * 16, 16)] = jnp.zeros((16,), jnp.float32)
        return 0

    lax.fori_loop(0, CHUNK, _zrow, 0)
    r0 = s * ROWS_PER_TILE
    nz = ROWS_PER_TILE // CHUNK
    zcps = []
    for z in range(nz):
        zcps.append(pltpu.async_copy(zb, ysh.at[pl.ds(r0 + z * CHUNK, CHUNK)], sem_i))
    rem = ROWS_PER_TILE - nz * CHUNK
    if rem:
        zcps.append(pltpu.async_copy(zb.at[pl.ds(0, rem)],
                                     ysh.at[pl.ds(r0 + nz * CHUNK, rem)], sem_i))

    @pl.when(s == 0)
    def _zero_tail():
        pltpu.async_copy(zb.at[pl.ds(0, ROWS_REM)],
                         ysh.at[pl.ds(NUM_TILES * ROWS_PER_TILE, ROWS_REM)],
                         sem_i).wait()

    for cp in zcps:
        cp.wait()
    plsc.subcore_barrier()

    def chunk_base(nn):
        return t0 + jnp.where(nn >= NFULL, TAIL_BASE, nn * CHUNK)

    def issue_idx(b, r):
        pltpu.async_copy(ii_hbm.at[pl.ds(b, CHUNK)], ii8.at[r], sem_i)
        pltpu.async_copy(ij_hbm.at[pl.ds(b, CHUNK)], ij8.at[r], sem_i)
        pltpu.async_copy(phi_hbm.at[pl.ds(b, CHUNK)], phib.at[r], sem_i)

    def wait_idx():
        pltpu.make_async_copy(ii_hbm.at[pl.ds(0, CHUNK)], ii8.at[0], sem_i).wait()
        pltpu.make_async_copy(ij_hbm.at[pl.ds(0, CHUNK)], ij8.at[0], sem_i).wait()
        pltpu.make_async_copy(phi_hbm.at[pl.ds(0, CHUNK)], phib.at[0], sem_i).wait()

    def issue_gathers(b, u, r):
        pltpu.async_copy(q_hbm.at[c].at[ii8.at[r]], qb3.at[u], gsem.at[u])
        pltpu.async_copy(k_hbm.at[c].at[ij8.at[r]], kb3.at[u], gsem.at[u])
        pltpu.async_copy(v_hbm.at[c].at[ij8.at[r]], vb3.at[u], gsem.at[u])
        pltpu.async_copy(w_hbm.at[pl.ds(b, CHUNK), pl.ds(c * FH, FH)], wb3.at[u],
                         gsem.at[u])

    def wait_gathers(u):
        pltpu.make_async_copy(q_hbm.at[c].at[ii8.at[0]], qb3.at[u], gsem.at[u]).wait()
        pltpu.make_async_copy(k_hbm.at[c].at[ij8.at[0]], kb3.at[u], gsem.at[u]).wait()
        pltpu.make_async_copy(v_hbm.at[c].at[ij8.at[0]], vb3.at[u], gsem.at[u]).wait()
        pltpu.make_async_copy(w_hbm.at[pl.ds(0, CHUNK), pl.ds(0, FH)], wb3.at[u],
                              gsem.at[u]).wait()

    def compute(u, r):
        @plsc.parallel_loop(0, NGROUP, 1)
        def _group(g):
            pv = phib[r, pl.ds(g * 16, 16)]
            for j in range(16):
                e = g * 16 + j
                acc = jnp.zeros((16,), jnp.float32)
                for t in range(NSUB):
                    sl = pl.ds(t * 16, 16)
                    acc = acc + qb3[u, e, sl] * wb3[u, e, sl] * kb3[u, e, sl]
                alpha = _lane_total(acc) * pv[j]
                for t in range(NSUB):
                    sl = pl.ds(t * 16, 16)
                    cb3[u, e, sl] = alpha * vb3[u, e, sl]

    def issue_scatter(u, r):
        pltpu.async_copy(cb3.at[u], ysh.at[ii8.at[r]], ssem.at[u], add=True)

    def wait_scatter(u):
        pltpu.make_async_copy(cb3.at[u], ysh.at[ii8.at[0]], ssem.at[u]).wait()

    pltpu.sync_copy(ii_hbm.at[pl.ds(t0, CHUNK)], ii8.at[0])
    pltpu.sync_copy(ij_hbm.at[pl.ds(t0, CHUNK)], ij8.at[0])
    pltpu.sync_copy(phi_hbm.at[pl.ds(t0, CHUNK)], phib.at[0])
    issue_gathers(t0, 0, 0)
    issue_idx(t0 + CHUNK, 1)

    def _step(n, _):
        u = lax.rem(n, 2)
        r = lax.rem(n, 8)
        wait_idx()
        issue_idx(chunk_base(n + 2), lax.rem(n + 2, 8))
        issue_gathers(chunk_base(n + 1), lax.rem(n + 1, 2), lax.rem(n + 1, 8))
        wait_gathers(u)

        @pl.when(n >= 2)
        def _ws():
            wait_scatter(u)

        compute(u, r)

        @pl.when(n == NFULL)
        def _mask_tail():
            def _zdup(rr, _):
                for t in range(NSUB):
                    cb3[u, rr, pl.ds(t * 16, 16)] = jnp.zeros((16,), jnp.float32)
                return 0

            lax.fori_loop(0, CHUNK - 16, _zdup, 0)

        issue_scatter(u, r)
        return 0

    lax.fori_loop(0, NCHUNKS, _step, 0)

    wait_idx()
    wait_gathers(1)
    wait_scatter(1)
    wait_scatter(0)
    plsc.subcore_barrier()

    done = 0
    while done < ROWS_PER_TILE:
        sz = min(128, ROWS_PER_TILE - done)
        pltpu.sync_copy(ysh.at[pl.ds(r0 + done, sz)],
                        out_hbm.at[pl.ds(r0 + done, sz), pl.ds(c * FH, FH)])
        done += sz

    @pl.when(s == 0)
    def _write_tail():
        tb = NUM_TILES * ROWS_PER_TILE
        pltpu.sync_copy(ysh.at[pl.ds(tb, ROWS_REM)],
                        out_hbm.at[pl.ds(tb, ROWS_REM), pl.ds(c * FH, FH)])


def _edge_kernel(q_all, k_all, v_all, w_ij, phi_s, ii, ij):
    mesh = plsc.VectorSubcoreMesh(core_axis_name="c", subcore_axis_name="s")
    run = functools.partial(
        pl.kernel,
        out_type=jax.ShapeDtypeStruct((N, F), jnp.float32),
        mesh=mesh,
        scratch_types=[
            pltpu.VMEM((8, CHUNK), jnp.int32),
            pltpu.VMEM((8, CHUNK), jnp.int32),
            pltpu.VMEM((8, CHUNK), jnp.float32),
            pltpu.VMEM((2, CHUNK, FH), jnp.float32),
            pltpu.VMEM((2, CHUNK, FH), jnp.float32),
            pltpu.VMEM((2, CHUNK, FH), jnp.float32),
            pltpu.VMEM((2, CHUNK, FH), jnp.float32),
            pltpu.VMEM((2, CHUNK, FH), jnp.float32),
            pltpu.VMEM_SHARED((N, FH), jnp.float32),
            pltpu.SemaphoreType.DMA,
            pltpu.SemaphoreType.DMA((2,)),
            pltpu.SemaphoreType.DMA((2,)),
        ],
    )(_edge_body)
    return run(q_all, k_all, v_all, w_ij, phi_s, ii, ij)


def kernel(x, w_ij, phi_r_cut, idx_i, idx_j, Wq, Wk, Wv):
    q_all, k_all, v_all = _qkv_project(x, Wq, Wk, Wv)
    phi_s = phi_r_cut[:, 0] * jnp.float32(1.0 / math.sqrt(FH))
    ii = idx_i.astype(jnp.int32)
    ij = idx_j.astype(jnp.int32)
    return _edge_kernel(q_all, k_all, v_all, w_ij, phi_s, ii, ij)
